# baseline (speedup 1.0000x reference)
import jax
import jax.numpy as jnp
from jax.experimental import pallas as pl
from jax.experimental.pallas import tpu as pltpu

_SIGMA = 0.5
_THRESH = 0.2
_N = 10000
_ROWS = 80
_LANES = 128
_PAD = _ROWS * _LANES
_K = 8
_NEG = -jnp.inf


def _nms_kernel(x1_ref, y1_ref, x2_ref, y2_ref, s_ref, tbl_ref, out_ref, v_ref):
    x1 = x1_ref[...]
    y1 = y1_ref[...]
    x2 = x2_ref[...]
    y2 = y2_ref[...]
    area = (x2 - x1) * (y2 - y1)
    s = s_ref[...]
    v_ref[...] = jnp.where(s > _THRESH, s, 0.0)

    rows = jax.lax.broadcasted_iota(jnp.int32, (_ROWS, _LANES), 0)
    cols = jax.lax.broadcasted_iota(jnp.int32, (_ROWS, _LANES), 1)
    flat = rows * _LANES + cols

    def body(c):
        i, _ = c
        v = v_ref[...]

        cum = v
        idxs, bxs = [], []
        for _j in range(_K):
            idx = jnp.argmax(cum)
            cum = jnp.where(flat == idx, _NEG, cum)
            row = tbl_ref[pl.ds(idx, 1), :]
            bx1 = row[:, 0:1]
            by1 = row[:, 1:2]
            bx2 = row[:, 2:3]
            by2 = row[:, 3:4]
            ba = (bx2 - bx1) * (by2 - by1)
            idxs.append(idx)
            bxs.append((bx1, by1, bx2, by2, ba))

        vals = []
        val = None
        keepmask = None
        for j in range(_K):
            ok = None
            aj = bxs[j]
            for i2 in range(j):
                ai = bxs[i2]
                ix = jnp.minimum(aj[2], ai[2]) - jnp.maximum(aj[0], ai[0])
                iy = jnp.minimum(aj[3], ai[3]) - jnp.maximum(aj[1], ai[1])
                ov = (ix > 0.0) & (iy > 0.0)
                ok = ov if ok is None else (ok | ov)
            if ok is None:
                nov = jnp.full((1, 1), True)
            else:
                nov = ~ok
            val = nov if val is None else (val & nov)
            vals.append(val)
            picked = val & (flat == idxs[j])
            keepmask = picked if keepmask is None else (keepmask | picked)

        newv = v
        for j in range(_K):
            bx1, by1, bx2, by2, ba = bxs[j]
            ix1 = jnp.maximum(bx1, x1)
            iy1 = jnp.maximum(by1, y1)
            ix2 = jnp.minimum(bx2, x2)
            iy2 = jnp.minimum(by2, y2)
            inter = jnp.maximum(ix2 - ix1, 0.0) * jnp.maximum(iy2 - iy1, 0.0)
            iou = inter / (ba + area - inter + 1e-9)
            e = jnp.where(
                vals[j] & (flat != idxs[j]), jnp.exp(-(iou * iou) / _SIGMA), 1.0
            )
            newv = newv * e
        newv = jnp.where(v > 0.0, newv, v)

        newv = jnp.where(keepmask & (newv > _THRESH), -newv, newv)

        v_ref[...] = newv
        return (i + 1, jnp.max(newv) > _THRESH)

    def cond(c):
        i, live = c
        return live & (i < _PAD)

    jax.lax.while_loop(cond, body, (jnp.int32(0), jnp.bool_(True)))
    out_ref[...] = v_ref[...]


@jax.jit
def kernel(results1, results2):
    results = jnp.concatenate([results1, results2], axis=0)
    box = results[:, 2:6]
    scores = results[:, 13]

    def col(vv):
        return jnp.pad(vv, (0, _PAD - _N)).reshape(_ROWS, _LANES)

    tbl = jnp.pad(box, ((0, _PAD - _N), (0, 0)))

    final2d = pl.pallas_call(
        _nms_kernel,
        out_shape=jax.ShapeDtypeStruct((_ROWS, _LANES), jnp.float32),
        scratch_shapes=[pltpu.VMEM((_ROWS, _LANES), jnp.float32)],
    )(
        col(box[:, 0]),
        col(box[:, 1]),
        col(box[:, 2]),
        col(box[:, 3]),
        col(scores),
        tbl,
    )

    vflat = final2d.reshape(_PAD)[:_N]
    keep = vflat < -_THRESH
    out = results.at[:, 13].set(-vflat)
    out = jnp.where(keep[:, None], out, 0.0)
    return out


# speedup vs baseline: 1.1154x; 1.1154x over previous
import jax
import jax.numpy as jnp
from jax.experimental import pallas as pl
from jax.experimental.pallas import tpu as pltpu

_SIGMA = 0.5
_THRESH = 0.2
_N = 10000
_ROWS = 80
_LANES = 128
_PAD = _ROWS * _LANES
_K = 8
_NEG = -jnp.inf


def _nms_kernel(x1_ref, y1_ref, x2_ref, y2_ref, s_ref, out_ref, v_ref):
    x1 = x1_ref[...]
    y1 = y1_ref[...]
    x2 = x2_ref[...]
    y2 = y2_ref[...]
    area = (x2 - x1) * (y2 - y1)
    s = s_ref[...]
    v_ref[...] = jnp.where(s > _THRESH, s, 0.0)

    rows = jax.lax.broadcasted_iota(jnp.int32, (_ROWS, _LANES), 0)
    cols = jax.lax.broadcasted_iota(jnp.int32, (_ROWS, _LANES), 1)
    flat = rows * _LANES + cols

    def extract(arr, oh):
        return jnp.sum(jnp.where(oh, arr, 0.0), axis=(0, 1), keepdims=True)

    def body(c):
        i, _ = c
        v = v_ref[...]

        cum = v
        idxs, bxs = [], []
        for _j in range(_K):
            idx = jnp.argmax(cum)
            oh = flat == idx
            cum = jnp.where(oh, _NEG, cum)
            idxs.append(idx)
            bxs.append(
                (
                    extract(x1, oh),
                    extract(y1, oh),
                    extract(x2, oh),
                    extract(y2, oh),
                    extract(area, oh),
                )
            )

        vals = []
        val = None
        keepmask = None
        for j in range(_K):
            ok = None
            aj = bxs[j]
            for i2 in range(j):
                ai = bxs[i2]
                ix = jnp.minimum(aj[2], ai[2]) - jnp.maximum(aj[0], ai[0])
                iy = jnp.minimum(aj[3], ai[3]) - jnp.maximum(aj[1], ai[1])
                ov = (ix > 0.0) & (iy > 0.0)
                ok = ov if ok is None else (ok | ov)
            if ok is None:
                nov = jnp.full((1, 1), True)
            else:
                nov = ~ok
            val = nov if val is None else (val & nov)
            vals.append(val)
            picked = val & (flat == idxs[j])
            keepmask = picked if keepmask is None else (keepmask | picked)

        newv = v
        for j in range(_K):
            bx1, by1, bx2, by2, ba = bxs[j]
            ix1 = jnp.maximum(bx1, x1)
            iy1 = jnp.maximum(by1, y1)
            ix2 = jnp.minimum(bx2, x2)
            iy2 = jnp.minimum(by2, y2)
            inter = jnp.maximum(ix2 - ix1, 0.0) * jnp.maximum(iy2 - iy1, 0.0)
            iou = inter / (ba + area - inter + 1e-9)
            e = jnp.where(
                vals[j] & (flat != idxs[j]), jnp.exp(-(iou * iou) / _SIGMA), 1.0
            )
            newv = newv * e
        newv = jnp.where(v > 0.0, newv, v)

        newv = jnp.where(keepmask & (newv > _THRESH), -newv, newv)

        v_ref[...] = newv
        return (i + 1, jnp.max(newv) > _THRESH)

    def cond(c):
        i, live = c
        return live & (i < _PAD)

    jax.lax.while_loop(cond, body, (jnp.int32(0), jnp.bool_(True)))
    out_ref[...] = v_ref[...]


@jax.jit
def kernel(results1, results2):
    results = jnp.concatenate([results1, results2], axis=0)
    box = results[:, 2:6]
    scores = results[:, 13]

    def col(vv):
        return jnp.pad(vv, (0, _PAD - _N)).reshape(_ROWS, _LANES)

    final2d = pl.pallas_call(
        _nms_kernel,
        out_shape=jax.ShapeDtypeStruct((_ROWS, _LANES), jnp.float32),
        scratch_shapes=[pltpu.VMEM((_ROWS, _LANES), jnp.float32)],
    )(
        col(box[:, 0]),
        col(box[:, 1]),
        col(box[:, 2]),
        col(box[:, 3]),
        col(scores),
    )

    vflat = final2d.reshape(_PAD)[:_N]
    keep = vflat < -_THRESH
    out = results.at[:, 13].set(-vflat)
    out = jnp.where(keep[:, None], out, 0.0)
    return out


# speedup vs baseline: 1.1476x; 1.0289x over previous
import jax
import jax.numpy as jnp
from jax.experimental import pallas as pl
from jax.experimental.pallas import tpu as pltpu

_SIGMA = 0.5
_THRESH = 0.2
_N = 10000
_ROWS = 80
_LANES = 128
_PAD = _ROWS * _LANES
_K = 8
_NEG = -jnp.inf


def _nms_kernel(x1_ref, y1_ref, x2_ref, y2_ref, s_ref, out_ref, v_ref):
    x1 = x1_ref[...]
    y1 = y1_ref[...]
    x2 = x2_ref[...]
    y2 = y2_ref[...]
    area = (x2 - x1) * (y2 - y1)
    s = s_ref[...]
    v_ref[...] = jnp.where(s > _THRESH, s, 0.0)

    rows = jax.lax.broadcasted_iota(jnp.int32, (_ROWS, _LANES), 0)
    cols = jax.lax.broadcasted_iota(jnp.int32, (_ROWS, _LANES), 1)
    flat = rows * _LANES + cols

    def extract(arr, oh):
        return jnp.sum(jnp.where(oh, arr, 0.0), axis=(0, 1), keepdims=True)

    def batch(v):
        cum = v
        idxs, bxs = [], []
        for _j in range(_K):
            idx = jnp.argmax(cum)
            oh = flat == idx
            cum = jnp.where(oh, _NEG, cum)
            idxs.append(idx)
            bxs.append(
                (
                    extract(x1, oh),
                    extract(y1, oh),
                    extract(x2, oh),
                    extract(y2, oh),
                    extract(area, oh),
                )
            )

        vals = []
        val = None
        keepmask = None
        for j in range(_K):
            ok = None
            aj = bxs[j]
            for i2 in range(j):
                ai = bxs[i2]
                ix = jnp.minimum(aj[2], ai[2]) - jnp.maximum(aj[0], ai[0])
                iy = jnp.minimum(aj[3], ai[3]) - jnp.maximum(aj[1], ai[1])
                ov = (ix > 0.0) & (iy > 0.0)
                ok = ov if ok is None else (ok | ov)
            if ok is None:
                nov = jnp.full((1, 1), True)
            else:
                nov = ~ok
            val = nov if val is None else (val & nov)
            vals.append(val)
            picked = val & (flat == idxs[j])
            keepmask = picked if keepmask is None else (keepmask | picked)

        newv = v
        for j in range(_K):
            bx1, by1, bx2, by2, ba = bxs[j]
            ix1 = jnp.maximum(bx1, x1)
            iy1 = jnp.maximum(by1, y1)
            ix2 = jnp.minimum(bx2, x2)
            iy2 = jnp.minimum(by2, y2)
            inter = jnp.maximum(ix2 - ix1, 0.0) * jnp.maximum(iy2 - iy1, 0.0)
            iou = inter / (ba + area - inter + 1e-9)
            e = jnp.where(
                vals[j] & (flat != idxs[j]), jnp.exp(-(iou * iou) / _SIGMA), 1.0
            )
            newv = newv * e
        newv = jnp.where(v > 0.0, newv, v)

        return jnp.where(keepmask & (newv > _THRESH), -newv, newv)

    def body(c):
        i, _ = c
        v = batch(batch(v_ref[...]))
        v_ref[...] = v
        return (i + 1, jnp.max(v) > _THRESH)

    def cond(c):
        i, live = c
        return live & (i < _PAD)

    jax.lax.while_loop(cond, body, (jnp.int32(0), jnp.bool_(True)))
    out_ref[...] = v_ref[...]


@jax.jit
def kernel(results1, results2):
    results = jnp.concatenate([results1, results2], axis=0)
    box = results[:, 2:6]
    scores = results[:, 13]

    def col(vv):
        return jnp.pad(vv, (0, _PAD - _N)).reshape(_ROWS, _LANES)

    final2d = pl.pallas_call(
        _nms_kernel,
        out_shape=jax.ShapeDtypeStruct((_ROWS, _LANES), jnp.float32),
        scratch_shapes=[pltpu.VMEM((_ROWS, _LANES), jnp.float32)],
    )(
        col(box[:, 0]),
        col(box[:, 1]),
        col(box[:, 2]),
        col(box[:, 3]),
        col(scores),
    )

    vflat = final2d.reshape(_PAD)[:_N]
    keep = vflat < -_THRESH
    out = results.at[:, 13].set(-vflat)
    out = jnp.where(keep[:, None], out, 0.0)
    return out


# speedup vs baseline: 1.1586x; 1.0096x over previous
import jax
import jax.numpy as jnp
from jax.experimental import pallas as pl
from jax.experimental.pallas import tpu as pltpu

_SIGMA = 0.5
_THRESH = 0.2
_N = 10000
_ROWS = 80
_LANES = 128
_PAD = _ROWS * _LANES
_K = 8
_NEG = -jnp.inf
_BIGF = 3.0e7


def _nms_kernel(x1_ref, y1_ref, x2_ref, y2_ref, s_ref, out_ref, v_ref, vs_ref, fs_ref):
    x1 = x1_ref[...]
    y1 = y1_ref[...]
    x2 = x2_ref[...]
    y2 = y2_ref[...]
    area = (x2 - x1) * (y2 - y1)
    s = s_ref[...]
    v_ref[...] = jnp.where(s > _THRESH, s, 0.0)
    vs_ref[...] = jnp.full((16, _LANES), _NEG, jnp.float32)
    fs_ref[...] = jnp.zeros((16, _LANES), jnp.float32)

    rows = jax.lax.broadcasted_iota(jnp.int32, (_ROWS, _LANES), 0)
    cols = jax.lax.broadcasted_iota(jnp.int32, (_ROWS, _LANES), 1)
    flatf = (cols * _ROWS + (rows % 8) * 10 + rows // 8).astype(jnp.float32)

    def cmb(a, b):
        gt = b[0] > a[0]
        return jnp.where(gt, b[0], a[0]), jnp.where(gt, b[1], a[1])

    def argmax11(cum):
        def f40(qv, qf):
            lo = cmb((qv[0:8], qf[0:8]), (qv[8:16], qf[8:16]))
            mid = cmb((qv[16:24], qf[16:24]), (qv[24:32], qf[24:32]))
            hi = cmb(mid, (qv[32:40], qf[32:40]))
            return cmb(lo, hi)

        v8, f8 = cmb(f40(cum[0:40], flatf[0:40]), f40(cum[40:80], flatf[40:80]))
        vs_ref[0:8, :] = v8
        fs_ref[0:8, :] = f8
        loads = [(vs_ref[k : k + 8, :], fs_ref[k : k + 8, :]) for k in range(8)]
        p0 = cmb(loads[0], loads[1])
        p1 = cmb(loads[2], loads[3])
        p2 = cmb(loads[4], loads[5])
        p3 = cmb(loads[6], loads[7])
        zv, zf = cmb(cmb(p0, p1), cmb(p2, p3))
        v0 = zv[0:1, :]
        f0 = zf[0:1, :]
        m = jnp.max(v0, axis=(0, 1), keepdims=True)
        fwin = jnp.min(
            jnp.where(v0 == m, f0, _BIGF), axis=(0, 1), keepdims=True
        )
        return m, fwin

    def extract(arr, oh):
        return jnp.sum(jnp.where(oh, arr, 0.0), axis=(0, 1), keepdims=True)

    def batch(v):
        cum = v
        fwins, bxs = [], []
        for _j in range(_K):
            _m, fwin = argmax11(cum)
            oh = flatf == fwin
            cum = jnp.where(oh, _NEG, cum)
            fwins.append(fwin)
            bxs.append(
                (
                    extract(x1, oh),
                    extract(y1, oh),
                    extract(x2, oh),
                    extract(y2, oh),
                    extract(area, oh),
                )
            )

        vals = []
        val = None
        keepmask = None
        for j in range(_K):
            ok = None
            aj = bxs[j]
            for i2 in range(j):
                ai = bxs[i2]
                ix = jnp.minimum(aj[2], ai[2]) - jnp.maximum(aj[0], ai[0])
                iy = jnp.minimum(aj[3], ai[3]) - jnp.maximum(aj[1], ai[1])
                ov = (ix > 0.0) & (iy > 0.0)
                ok = ov if ok is None else (ok | ov)
            if ok is None:
                nov = jnp.full((1, 1), True)
            else:
                nov = ~ok
            val = nov if val is None else (val & nov)
            vals.append(val)
            picked = val & (flatf == fwins[j])
            keepmask = picked if keepmask is None else (keepmask | picked)

        newv = v
        for j in range(_K):
            bx1, by1, bx2, by2, ba = bxs[j]
            ix1 = jnp.maximum(bx1, x1)
            iy1 = jnp.maximum(by1, y1)
            ix2 = jnp.minimum(bx2, x2)
            iy2 = jnp.minimum(by2, y2)
            inter = jnp.maximum(ix2 - ix1, 0.0) * jnp.maximum(iy2 - iy1, 0.0)
            iou = inter / (ba + area - inter + 1e-9)
            e = jnp.where(
                vals[j] & (flatf != fwins[j]),
                jnp.exp(-(iou * iou) / _SIGMA),
                1.0,
            )
            newv = newv * e
        newv = jnp.where(v > 0.0, newv, v)

        return jnp.where(keepmask & (newv > _THRESH), -newv, newv)

    def body(c):
        i, _ = c
        v = batch(batch(v_ref[...]))
        v_ref[...] = v
        return (i + 1, jnp.max(v) > _THRESH)

    def cond(c):
        i, live = c
        return live & (i < _PAD)

    jax.lax.while_loop(cond, body, (jnp.int32(0), jnp.bool_(True)))
    out_ref[...] = v_ref[...]


@jax.jit
def kernel(results1, results2):
    results = jnp.concatenate([results1, results2], axis=0)
    box = results[:, 2:6]
    scores = results[:, 13]

    p_of_row = [(r % 8) * 10 + r // 8 for r in range(_ROWS)]
    row_of_p = [8 * (p % 10) + p // 10 for p in range(_ROWS)]

    def col(vv):
        x = jnp.pad(vv, (0, _PAD - _N)).reshape(_LANES, _ROWS)
        return jnp.transpose(x[:, jnp.array(p_of_row)])

    final2d = pl.pallas_call(
        _nms_kernel,
        out_shape=jax.ShapeDtypeStruct((_ROWS, _LANES), jnp.float32),
        scratch_shapes=[
            pltpu.VMEM((_ROWS, _LANES), jnp.float32),
            pltpu.VMEM((16, _LANES), jnp.float32),
            pltpu.VMEM((16, _LANES), jnp.float32),
        ],
    )(col(box[:, 0]), col(box[:, 1]), col(box[:, 2]), col(box[:, 3]), col(scores))

    vflat = jnp.transpose(final2d[jnp.array(row_of_p), :]).reshape(_PAD)[:_N]
    keep = vflat < -_THRESH
    out = results.at[:, 13].set(-vflat)
    out = jnp.where(keep[:, None], out, 0.0)
    return out


# speedup vs baseline: 1.1884x; 1.0257x over previous
import jax
import jax.numpy as jnp
from jax.experimental import pallas as pl
from jax.experimental.pallas import tpu as pltpu

_SIGMA = 0.5
_THRESH = 0.2
_N = 10000
_ROWS = 80
_LANES = 128
_PAD = _ROWS * _LANES
_K = 8
_NEG = -jnp.inf
_BIGF = 3.0e7


def _nms_kernel(x1_ref, y1_ref, x2_ref, y2_ref, s_ref, out_ref, v_ref, vs_ref, fs_ref):
    x1 = x1_ref[...]
    y1 = y1_ref[...]
    x2 = x2_ref[...]
    y2 = y2_ref[...]
    area = (x2 - x1) * (y2 - y1)
    s = s_ref[...]
    v_ref[...] = jnp.where(s > _THRESH, s, 0.0)
    vs_ref[...] = jnp.full((16, _LANES), _NEG, jnp.float32)
    fs_ref[...] = jnp.zeros((16, _LANES), jnp.float32)

    rows = jax.lax.broadcasted_iota(jnp.int32, (_ROWS, _LANES), 0)
    cols = jax.lax.broadcasted_iota(jnp.int32, (_ROWS, _LANES), 1)
    flatf = (cols * _ROWS + (rows % 8) * 10 + rows // 8).astype(jnp.float32)

    def cmb(a, b):
        gt = b[0] > a[0]
        return jnp.where(gt, b[0], a[0]), jnp.where(gt, b[1], a[1])

    def argmax11(cum):
        def f40(qv, qf):
            lo = cmb((qv[0:8], qf[0:8]), (qv[8:16], qf[8:16]))
            mid = cmb((qv[16:24], qf[16:24]), (qv[24:32], qf[24:32]))
            hi = cmb(mid, (qv[32:40], qf[32:40]))
            return cmb(lo, hi)

        v8, f8 = cmb(f40(cum[0:40], flatf[0:40]), f40(cum[40:80], flatf[40:80]))
        vs_ref[0:8, :] = v8
        fs_ref[0:8, :] = f8
        loads = [(vs_ref[k : k + 8, :], fs_ref[k : k + 8, :]) for k in range(8)]
        p0 = cmb(loads[0], loads[1])
        p1 = cmb(loads[2], loads[3])
        p2 = cmb(loads[4], loads[5])
        p3 = cmb(loads[6], loads[7])
        zv, zf = cmb(cmb(p0, p1), cmb(p2, p3))
        v0 = zv[0:1, :]
        f0 = zf[0:1, :]
        m = jnp.max(v0, axis=(0, 1), keepdims=True)
        fwin = jnp.min(
            jnp.where(v0 == m, f0, _BIGF), axis=(0, 1), keepdims=True
        )
        return m, fwin

    def extract(arr, oh):
        return jnp.sum(jnp.where(oh, arr, 0.0), axis=(0, 1), keepdims=True)

    def batch(v):
        cum = v
        fwins, bxs = [], []
        for _j in range(_K):
            _m, fwin = argmax11(cum)
            oh = flatf == fwin
            cum = jnp.where(oh, _NEG, cum)
            fwins.append(fwin)
            bx1 = extract(x1, oh)
            by1 = extract(y1, oh)
            bx2 = extract(x2, oh)
            by2 = extract(y2, oh)
            bxs.append((bx1, by1, bx2, by2, (bx2 - bx1) * (by2 - by1)))

        vals = []
        val = None
        keepmask = None
        for j in range(_K):
            ok = None
            aj = bxs[j]
            for i2 in range(j):
                ai = bxs[i2]
                ix = jnp.minimum(aj[2], ai[2]) - jnp.maximum(aj[0], ai[0])
                iy = jnp.minimum(aj[3], ai[3]) - jnp.maximum(aj[1], ai[1])
                ov = (ix > 0.0) & (iy > 0.0)
                ok = ov if ok is None else (ok | ov)
            if ok is None:
                nov = jnp.full((1, 1), True)
            else:
                nov = ~ok
            val = nov if val is None else (val & nov)
            vals.append(val)
            picked = val & (flatf == fwins[j])
            keepmask = picked if keepmask is None else (keepmask | picked)

        newv = v
        for j in range(_K):
            bx1, by1, bx2, by2, ba = bxs[j]
            ix1 = jnp.maximum(bx1, x1)
            iy1 = jnp.maximum(by1, y1)
            ix2 = jnp.minimum(bx2, x2)
            iy2 = jnp.minimum(by2, y2)
            inter = jnp.maximum(ix2 - ix1, 0.0) * jnp.maximum(iy2 - iy1, 0.0)
            iou = inter / (ba + area - inter + 1e-9)
            e = jnp.where(
                vals[j] & (flatf != fwins[j]),
                jnp.exp(-(iou * iou) / _SIGMA),
                1.0,
            )
            newv = newv * e
        newv = jnp.where(v > 0.0, newv, v)

        return jnp.where(keepmask & (newv > _THRESH), -newv, newv)

    def body(c):
        i, _ = c
        v = batch(batch(v_ref[...]))
        v_ref[...] = v
        return (i + 1, jnp.max(v) > _THRESH)

    def cond(c):
        i, live = c
        return live & (i < _PAD)

    jax.lax.while_loop(cond, body, (jnp.int32(0), jnp.bool_(True)))
    out_ref[...] = v_ref[...]


@jax.jit
def kernel(results1, results2):
    results = jnp.concatenate([results1, results2], axis=0)
    box = results[:, 2:6]
    scores = results[:, 13]

    p_of_row = [(r % 8) * 10 + r // 8 for r in range(_ROWS)]
    row_of_p = [8 * (p % 10) + p // 10 for p in range(_ROWS)]

    def col(vv):
        x = jnp.pad(vv, (0, _PAD - _N)).reshape(_LANES, _ROWS)
        return jnp.transpose(x[:, jnp.array(p_of_row)])

    final2d = pl.pallas_call(
        _nms_kernel,
        out_shape=jax.ShapeDtypeStruct((_ROWS, _LANES), jnp.float32),
        scratch_shapes=[
            pltpu.VMEM((_ROWS, _LANES), jnp.float32),
            pltpu.VMEM((16, _LANES), jnp.float32),
            pltpu.VMEM((16, _LANES), jnp.float32),
        ],
    )(col(box[:, 0]), col(box[:, 1]), col(box[:, 2]), col(box[:, 3]), col(scores))

    vflat = jnp.transpose(final2d[jnp.array(row_of_p), :]).reshape(_PAD)[:_N]
    keep = vflat < -_THRESH
    out = results.at[:, 13].set(-vflat)
    out = jnp.where(keep[:, None], out, 0.0)
    return out
